# baseline (speedup 1.0000x reference)
import functools

import jax
import jax.numpy as jnp
from jax import lax
from jax.experimental import pallas as pl
from jax.experimental.pallas import tpu as pltpu
from jax.experimental.pallas import tpu_sc as plsc

_E = 10
_K = 3
_EP = 16
_BT = 2048


def _gating_body(x_ref, w_ref, b_ref, nb_ref, logits_ref, lpad_ref):
    y = jnp.dot(x_ref[...], w_ref[...], preferred_element_type=jnp.float32)
    y = y + b_ref[...]
    noise_std = jax.nn.softplus(y[:, _EP:])
    logits = y[:, :_EP] + nb_ref[...] * noise_std
    logits_ref[...] = logits[:, :_E]
    lpad_ref[...] = logits


def _gating(x, nb, Wc, bc):
    T, D = x.shape
    grid = (T // _BT,)
    return pl.pallas_call(
        _gating_body,
        grid=grid,
        in_specs=[
            pl.BlockSpec((_BT, D), lambda i: (i, 0)),
            pl.BlockSpec((D, 2 * _EP), lambda i: (0, 0)),
            pl.BlockSpec((1, 2 * _EP), lambda i: (0, 0)),
            pl.BlockSpec((_BT, _EP), lambda i: (i, 0)),
        ],
        out_specs=[
            pl.BlockSpec((_BT, _E), lambda i: (i, 0)),
            pl.BlockSpec((_BT, _EP), lambda i: (i, 0)),
        ],
        out_shape=[
            jax.ShapeDtypeStruct((T, _E), jnp.float32),
            jax.ShapeDtypeStruct((T, _EP), jnp.float32),
        ],
        compiler_params=pltpu.CompilerParams(
            dimension_semantics=("arbitrary",),
        ),
    )(x, Wc, bc, nb)


def _make_sc_router(T, nc, ns):
    nw = nc * ns
    tpw = T // nw
    mesh = plsc.VectorSubcoreMesh(core_axis_name="c", subcore_axis_name="s")

    @functools.partial(
        pl.kernel,
        out_type=[
            jax.ShapeDtypeStruct((T * _E,), jnp.float32),
            jax.ShapeDtypeStruct((T * _E,), jnp.float32),
            jax.ShapeDtypeStruct((T * _K,), jnp.int32),
        ],
        mesh=mesh,
        scratch_types=[
            pltpu.VMEM((tpw * _EP,), jnp.float32),
            pltpu.VMEM((tpw * _E + _EP,), jnp.float32),
            pltpu.VMEM((tpw * _E + _EP,), jnp.float32),
            pltpu.VMEM((tpw * _K + _EP,), jnp.int32),
            pltpu.VMEM((_EP,), jnp.int32),
        ],
        compiler_params=pltpu.CompilerParams(needs_layout_passes=False),
    )
    def sc_router(lpad_hbm, keep_hbm, w_hbm, d_hbm, i_hbm,
                  in_v, w_v, d_v, i_v, k_v):
        wid = lax.axis_index("s") * nc + lax.axis_index("c")
        pltpu.sync_copy(lpad_hbm.at[pl.ds(wid * tpw * _EP, tpw * _EP)], in_v)
        pltpu.sync_copy(keep_hbm, k_v)
        lanes = lax.iota(jnp.int32, _EP)
        keepb = k_v[...] != 0
        emask = lanes < _E
        kmask = lanes < _K
        zeros = jnp.zeros((_EP,), jnp.float32)
        perms = [lanes ^ (1 << j) for j in range(4)]

        def _bcast_max(v):
            for p in perms:
                v = jnp.maximum(v, jnp.take(v, p))
            return v

        def _bcast_sum(v):
            for p in perms:
                v = v + jnp.take(v, p)
            return v

        def body(t, carry):
            lg = in_v[pl.ds(t * _EP, _EP)]
            m = _bcast_max(lg)
            ex = jnp.exp(lg - m)
            w = ex / _bcast_sum(ex)
            plsc.store_scatter(w_v, [t * _E + lanes], w, mask=emask)
            wk = jnp.where(emask, w, -1.0)
            sk, sv = plsc.sort_key_val(wk, lanes, descending=True)
            kept = jnp.where(keepb, sk, 0.0)
            norm = kept / _bcast_sum(kept)
            plsc.store_scatter(d_v, [t * _E + lanes], zeros, mask=emask)
            plsc.store_scatter(d_v, [t * _E + sv], norm, mask=keepb)
            plsc.store_scatter(i_v, [t * _K + lanes], sv, mask=kmask)
            return carry

        lax.fori_loop(0, tpw, body, 0)
        pltpu.sync_copy(w_v.at[pl.ds(0, tpw * _E)],
                        w_hbm.at[pl.ds(wid * tpw * _E, tpw * _E)])
        pltpu.sync_copy(d_v.at[pl.ds(0, tpw * _E)],
                        d_hbm.at[pl.ds(wid * tpw * _E, tpw * _E)])
        pltpu.sync_copy(i_v.at[pl.ds(0, tpw * _K)],
                        i_hbm.at[pl.ds(wid * tpw * _K, tpw * _K)])

    return sc_router


def kernel(x, noise_base, Wg, bg, Wn, bn, top_k):
    T, D = x.shape
    E = Wg.shape[0]
    Wc = (jnp.zeros((D, 2 * _EP), jnp.float32)
          .at[:, :E].set(Wg.T).at[:, _EP:_EP + E].set(Wn.T))
    bc = (jnp.full((1, 2 * _EP), 0.0, jnp.float32)
          .at[0, :E].set(bg)
          .at[0, E:_EP].set(-1e30)
          .at[0, _EP:_EP + E].set(bn))
    nb = jnp.pad(noise_base, ((0, 0), (0, _EP - E)))
    keep = (jnp.arange(_EP, dtype=jnp.int32)
            < jnp.minimum(jnp.asarray(top_k, jnp.int32), _K)).astype(jnp.int32)

    logits, lpad = _gating(x, nb, Wc, bc)

    info = plsc.get_sparse_core_info()
    router = _make_sc_router(T, info.num_cores, info.num_subcores)
    w_flat, d_flat, i_flat = router(lpad.reshape(-1), keep)

    weights = w_flat.reshape(T, E)
    dense = d_flat.reshape(T, E)
    idx = i_flat.reshape(T, _K)
    return (dense, weights, logits, idx)


# speedup vs baseline: 1.2380x; 1.2380x over previous
import functools

import jax
import jax.numpy as jnp
from jax import lax
from jax.experimental import pallas as pl
from jax.experimental.pallas import tpu as pltpu
from jax.experimental.pallas import tpu_sc as plsc

_E = 10
_K = 3
_EP = 16
_BT = 2048


def _gating_body(x_ref, w_ref, b_ref, nb_ref, logits_ref, weights_ref, wpad_ref):
    y = jnp.dot(x_ref[...], w_ref[...], preferred_element_type=jnp.float32)
    y = y + b_ref[...]
    noise_std = jax.nn.softplus(y[:, _EP:])
    logits = y[:, :_EP] + nb_ref[...] * noise_std
    logits_ref[...] = logits[:, :_E]
    m = jnp.max(logits, axis=1, keepdims=True)
    e = jnp.exp(logits - m)
    w = e / jnp.sum(e, axis=1, keepdims=True)
    weights_ref[...] = w[:, :_E]
    wpad_ref[...] = w


def _gating(x, nb, Wc, bc):
    T, D = x.shape
    grid = (T // _BT,)
    return pl.pallas_call(
        _gating_body,
        grid=grid,
        in_specs=[
            pl.BlockSpec((_BT, D), lambda i: (i, 0)),
            pl.BlockSpec((D, 2 * _EP), lambda i: (0, 0)),
            pl.BlockSpec((1, 2 * _EP), lambda i: (0, 0)),
            pl.BlockSpec((_BT, _EP), lambda i: (i, 0)),
        ],
        out_specs=[
            pl.BlockSpec((_BT, _E), lambda i: (i, 0)),
            pl.BlockSpec((_BT, _E), lambda i: (i, 0)),
            pl.BlockSpec((_BT, _EP), lambda i: (i, 0)),
        ],
        out_shape=[
            jax.ShapeDtypeStruct((T, _E), jnp.float32),
            jax.ShapeDtypeStruct((T, _E), jnp.float32),
            jax.ShapeDtypeStruct((T, _EP), jnp.float32),
        ],
        compiler_params=pltpu.CompilerParams(
            dimension_semantics=("arbitrary",),
        ),
    )(x, Wc, bc, nb)


def _make_sc_router(T, nc, ns):
    nw = nc * ns
    tpw = T // nw
    dlen = tpw * _E + _EP
    mesh = plsc.VectorSubcoreMesh(core_axis_name="c", subcore_axis_name="s")

    @functools.partial(
        pl.kernel,
        out_type=[
            jax.ShapeDtypeStruct((T * _E,), jnp.float32),
            jax.ShapeDtypeStruct((T * _K,), jnp.int32),
        ],
        mesh=mesh,
        scratch_types=[
            pltpu.VMEM((tpw * _EP,), jnp.float32),
            pltpu.VMEM((dlen,), jnp.float32),
            pltpu.VMEM((tpw * _K + _EP,), jnp.int32),
            pltpu.VMEM((_EP,), jnp.int32),
        ],
        compiler_params=pltpu.CompilerParams(needs_layout_passes=False),
    )
    def sc_router(wpad_hbm, keep_hbm, d_hbm, i_hbm, in_v, d_v, i_v, k_v):
        wid = lax.axis_index("s") * nc + lax.axis_index("c")
        pltpu.sync_copy(wpad_hbm.at[pl.ds(wid * tpw * _EP, tpw * _EP)], in_v)
        pltpu.sync_copy(keep_hbm, k_v)
        lanes = lax.iota(jnp.int32, _EP)
        keepb = k_v[...] != 0
        emask = lanes < _E
        kmask = lanes < _K
        zeros = jnp.zeros((_EP,), jnp.float32)
        p1 = lanes ^ 1
        p2 = lanes ^ 2

        @plsc.parallel_loop(0, dlen // _EP, unroll=8)
        def _zero(z):
            d_v[pl.ds(z * _EP, _EP)] = zeros

        @plsc.parallel_loop(0, tpw, unroll=4)
        def _route(t):
            w = in_v[pl.ds(t * _EP, _EP)]
            wk = jnp.where(emask, w, -1.0)
            sk, sv = plsc.sort_key_val(wk, lanes, descending=True)
            kept = jnp.where(keepb, sk, 0.0)
            s = kept + jnp.take(kept, p1)
            s = s + jnp.take(s, p2)
            norm = kept / s
            plsc.store_scatter(d_v, [t * _E + sv], norm, mask=keepb)
            plsc.store_scatter(i_v, [t * _K + lanes], sv, mask=kmask)

        pltpu.sync_copy(d_v.at[pl.ds(0, tpw * _E)],
                        d_hbm.at[pl.ds(wid * tpw * _E, tpw * _E)])
        pltpu.sync_copy(i_v.at[pl.ds(0, tpw * _K)],
                        i_hbm.at[pl.ds(wid * tpw * _K, tpw * _K)])

    return sc_router


def kernel(x, noise_base, Wg, bg, Wn, bn, top_k):
    T, D = x.shape
    E = Wg.shape[0]
    Wc = (jnp.zeros((D, 2 * _EP), jnp.float32)
          .at[:, :E].set(Wg.T).at[:, _EP:_EP + E].set(Wn.T))
    bc = (jnp.full((1, 2 * _EP), 0.0, jnp.float32)
          .at[0, :E].set(bg)
          .at[0, E:_EP].set(-1e30)
          .at[0, _EP:_EP + E].set(bn))
    nb = jnp.pad(noise_base, ((0, 0), (0, _EP - E)))
    keep = (jnp.arange(_EP, dtype=jnp.int32)
            < jnp.minimum(jnp.asarray(top_k, jnp.int32), _K)).astype(jnp.int32)

    logits, weights, wpad = _gating(x, nb, Wc, bc)

    info = plsc.get_sparse_core_info()
    router = _make_sc_router(T, info.num_cores, info.num_subcores)
    d_flat, i_flat = router(wpad.reshape(-1), keep)

    dense = d_flat.reshape(T, E)
    idx = i_flat.reshape(T, _K)
    return (dense, weights, logits, idx)
